# baseline (speedup 1.0000x reference)
import jax
import jax.numpy as jnp
from jax.experimental import pallas as pl
from jax.experimental.pallas import tpu as pltpu

_B, _S, _E, _D, _M = 4, 2048, 1024, 512, 64
_T = 512
_C = 128


def _dot_t(a, b):
    return jax.lax.dot_general(a, b, (((1,), (1,)), ((), ())),
                               preferred_element_type=jnp.float32)


def _body(emb_ref, prev_ref, wq_ref, bq_ref, ww_ref, bw_ref, wo_ref, bo_ref,
          out_ref):
    t0 = pl.program_id(1) * _T
    bf = jnp.bfloat16
    emb = emb_ref[0].astype(bf)
    prev = prev_ref[0, 0].astype(bf)
    wq = wq_ref[...].astype(bf)
    ww = ww_ref[...].astype(bf)

    q = (_dot_t(emb, wq) + bq_ref[...]).astype(bf)
    kc = _dot_t(emb, ww) + bw_ref[...]
    kp = _dot_t(prev, ww) + bw_ref[...]
    k = jnp.concatenate([kp, kc], axis=0).astype(bf)

    ii = jax.lax.broadcasted_iota(jnp.int32, (_C, _C + _M), 0)
    jj = jax.lax.broadcasted_iota(jnp.int32, (_C, _C + _M), 1)
    band = (jj >= ii) & (jj < ii + _M)
    parts = []
    for c in range(_T // _C):
        qc = q[c * _C:(c + 1) * _C]
        kcs = k[c * _C: c * _C + _C + _M]
        s = _dot_t(qc, kcs) * (_D ** -0.5)
        valid = band & (jj + t0 + c * _C >= _M)
        s = jnp.where(valid, s, jnp.float32(-1e30))
        mrow = jnp.max(s, axis=1, keepdims=True)
        e = jnp.where(valid, jnp.exp(s - mrow), 0.0)
        denom = jnp.sum(e, axis=1, keepdims=True)
        attn = (e / jnp.maximum(denom, jnp.float32(1e-30))).astype(bf)
        parts.append(jnp.dot(attn, kcs, preferred_element_type=jnp.float32))
    retrieved = jnp.concatenate(parts, axis=0)
    out_ref[0] = _dot_t(retrieved.astype(bf), wo_ref[...].astype(bf)) + bo_ref[...]


def kernel(embeddings, Ww, bw, Wq, bq, Wo, bo):
    nblk = _S // _T
    cpb = _T // _M
    emb4 = embeddings.reshape(_B, _S // _M, _M, _E)
    return pl.pallas_call(
        _body,
        grid=(_B, nblk),
        in_specs=[
            pl.BlockSpec((1, _T, _E), lambda b, i: (b, i, 0)),
            pl.BlockSpec((1, 1, _M, _E),
                         lambda b, i: (b, jnp.maximum(i * cpb - 1, 0), 0, 0)),
            pl.BlockSpec((_D, _E), lambda b, i: (0, 0)),
            pl.BlockSpec((1, _D), lambda b, i: (0, 0)),
            pl.BlockSpec((_D, _E), lambda b, i: (0, 0)),
            pl.BlockSpec((1, _D), lambda b, i: (0, 0)),
            pl.BlockSpec((_E, _D), lambda b, i: (0, 0)),
            pl.BlockSpec((1, _E), lambda b, i: (0, 0)),
        ],
        out_specs=pl.BlockSpec((1, _T, _E), lambda b, i: (b, i, 0)),
        out_shape=jax.ShapeDtypeStruct((_B, _S, _E), jnp.float32),
        compiler_params=pltpu.CompilerParams(
            dimension_semantics=("parallel", "parallel")),
    )(embeddings, emb4, Wq, bq.reshape(1, _D), Ww,
      bw.reshape(1, _D), Wo, bo.reshape(1, _E))


# speedup vs baseline: 1.0686x; 1.0686x over previous
import jax
import jax.numpy as jnp
from jax.experimental import pallas as pl
from jax.experimental.pallas import tpu as pltpu

_B, _S, _E, _D, _M = 4, 2048, 1024, 512, 64
_T = 512


def _dot_t(a, b):
    return jax.lax.dot_general(a, b, (((1,), (1,)), ((), ())),
                               preferred_element_type=jnp.float32)


def _body(emb_ref, prev_ref, wq_ref, bq_ref, ww_ref, bw_ref, wo_ref, bo_ref,
          out_ref):
    t0 = pl.program_id(1) * _T
    bf = jnp.bfloat16
    emb = emb_ref[0].astype(bf)
    prev = prev_ref[0, 0].astype(bf)
    wq = wq_ref[...].astype(bf)
    ww = ww_ref[...].astype(bf)

    q = (_dot_t(emb, wq) + bq_ref[...]).astype(bf)
    kc = _dot_t(emb, ww) + bw_ref[...]
    kp = _dot_t(prev, ww) + bw_ref[...]
    k = jnp.concatenate([kp, kc], axis=0).astype(bf)

    s = _dot_t(q, k) * (_D ** -0.5)
    ii = jax.lax.broadcasted_iota(jnp.int32, (_T, _T + _M), 0)
    jj = jax.lax.broadcasted_iota(jnp.int32, (_T, _T + _M), 1)
    valid = (jj >= ii) & (jj < ii + _M) & (jj + t0 >= _M)
    s = jnp.where(valid, s, jnp.float32(-1e30))
    mrow = jnp.max(s, axis=1, keepdims=True)
    e = jnp.where(valid, jnp.exp(s - mrow), 0.0)
    denom = jnp.sum(e, axis=1, keepdims=True)
    attn = (e / jnp.maximum(denom, jnp.float32(1e-30))).astype(bf)
    retrieved = jnp.dot(attn, k, preferred_element_type=jnp.float32)
    out_ref[0] = _dot_t(retrieved.astype(bf), wo_ref[...].astype(bf)) + bo_ref[...]


def kernel(embeddings, Ww, bw, Wq, bq, Wo, bo):
    nblk = _S // _T
    cpb = _T // _M
    emb4 = embeddings.reshape(_B, _S // _M, _M, _E)
    return pl.pallas_call(
        _body,
        grid=(_B, nblk),
        in_specs=[
            pl.BlockSpec((1, _T, _E), lambda b, i: (b, i, 0)),
            pl.BlockSpec((1, 1, _M, _E),
                         lambda b, i: (b, jnp.maximum(i * cpb - 1, 0), 0, 0)),
            pl.BlockSpec((_D, _E), lambda b, i: (0, 0)),
            pl.BlockSpec((1, _D), lambda b, i: (0, 0)),
            pl.BlockSpec((_D, _E), lambda b, i: (0, 0)),
            pl.BlockSpec((1, _D), lambda b, i: (0, 0)),
            pl.BlockSpec((_E, _D), lambda b, i: (0, 0)),
            pl.BlockSpec((1, _E), lambda b, i: (0, 0)),
        ],
        out_specs=pl.BlockSpec((1, _T, _E), lambda b, i: (b, i, 0)),
        out_shape=jax.ShapeDtypeStruct((_B, _S, _E), jnp.float32),
        compiler_params=pltpu.CompilerParams(
            dimension_semantics=("parallel", "parallel")),
    )(embeddings, emb4, Wq, bq.reshape(1, _D), Ww,
      bw.reshape(1, _D), Wo, bo.reshape(1, _E))


# speedup vs baseline: 1.1873x; 1.1111x over previous
import jax
import jax.numpy as jnp
from jax.experimental import pallas as pl
from jax.experimental.pallas import tpu as pltpu

_B, _S, _E, _D, _M = 4, 2048, 1024, 512, 64
_T = 512


def _dot_t(a, b):
    return jax.lax.dot_general(a, b, (((1,), (1,)), ((), ())),
                               preferred_element_type=jnp.float32)


def _body(emb_ref, wq_ref, bq_ref, ww_ref, bw_ref, wo_ref, bo_ref,
          out_ref, ktail_ref):
    i = pl.program_id(1)
    t0 = i * _T
    bf = jnp.bfloat16
    emb = emb_ref[0].astype(bf)
    wq = wq_ref[...].astype(bf)
    ww = ww_ref[...].astype(bf)

    q = (_dot_t(emb, wq) + bq_ref[...]).astype(bf)
    kc = (_dot_t(emb, ww) + bw_ref[...]).astype(bf)
    kp = jnp.where(i == 0, jnp.zeros_like(ktail_ref[...]), ktail_ref[...])
    k = jnp.concatenate([kp, kc], axis=0)

    s = _dot_t(q, k) * (_D ** -0.5)
    ii = jax.lax.broadcasted_iota(jnp.int32, (_T, _T + _M), 0)
    jj = jax.lax.broadcasted_iota(jnp.int32, (_T, _T + _M), 1)
    valid = (jj >= ii) & (jj < ii + _M) & (jj + t0 >= _M)
    s = jnp.where(valid, s, jnp.float32(-1e30))
    mrow = jnp.max(s, axis=1, keepdims=True)
    e = jnp.where(valid, jnp.exp(s - mrow), 0.0)
    denom = jnp.sum(e, axis=1, keepdims=True)
    attn = (e / jnp.maximum(denom, jnp.float32(1e-30))).astype(bf)
    retrieved = jnp.dot(attn, k, preferred_element_type=jnp.float32)
    out_ref[0] = _dot_t(retrieved.astype(bf), wo_ref[...].astype(bf)) + bo_ref[...]
    ktail_ref[...] = kc[_T - _M:, :]


def kernel(embeddings, Ww, bw, Wq, bq, Wo, bo):
    nblk = _S // _T
    return pl.pallas_call(
        _body,
        grid=(_B, nblk),
        in_specs=[
            pl.BlockSpec((1, _T, _E), lambda b, i: (b, i, 0)),
            pl.BlockSpec((_D, _E), lambda b, i: (0, 0)),
            pl.BlockSpec((1, _D), lambda b, i: (0, 0)),
            pl.BlockSpec((_D, _E), lambda b, i: (0, 0)),
            pl.BlockSpec((1, _D), lambda b, i: (0, 0)),
            pl.BlockSpec((_E, _D), lambda b, i: (0, 0)),
            pl.BlockSpec((1, _E), lambda b, i: (0, 0)),
        ],
        out_specs=pl.BlockSpec((1, _T, _E), lambda b, i: (b, i, 0)),
        out_shape=jax.ShapeDtypeStruct((_B, _S, _E), jnp.float32),
        scratch_shapes=[pltpu.VMEM((_M, _D), jnp.bfloat16)],
        compiler_params=pltpu.CompilerParams(
            dimension_semantics=("parallel", "arbitrary")),
    )(embeddings, Wq, bq.reshape(1, _D), Ww,
      bw.reshape(1, _D), Wo, bo.reshape(1, _E))
